# Initial kernel scaffold; baseline (speedup 1.0000x reference)
import jax
import jax.numpy as jnp
from jax.experimental import pallas as pl


def kernel(px, idx_i, idx_j):
    raise NotImplementedError("write your pallas kernel here")


# speedup vs baseline: 2.4571x; 2.4571x over previous
import functools
import jax
import jax.numpy as jnp
from jax import lax
from jax.experimental import pallas as pl
from jax.experimental.pallas import tpu as pltpu
from jax.experimental.pallas import tpu_sc as plsc

B = 320000
D = 128
NC = 2
NS = 16
NW = NC * NS
EPW = B // NW
C = 80
NCHUNK = EPW // C


def _sc_body(px_hbm, ii_hbm, jj_hbm, out_hbm, ii_v, jj_v, ri_v, rj_v,
             sem_i, sem_j):
    wid = lax.axis_index("s") * NC + lax.axis_index("c")
    base = wid * EPW

    pltpu.sync_copy(ii_hbm.at[wid], ii_v)
    pltpu.sync_copy(jj_hbm.at[wid], jj_v)

    def chunk(g, carry):
        cp_i = pltpu.async_copy(px_hbm.at[ii_v.at[g]], ri_v, sem_i)
        cp_j = pltpu.async_copy(px_hbm.at[jj_v.at[g]], rj_v, sem_j)
        cp_i.wait()
        cp_j.wait()

        def row(r, rc):
            for c8 in range(D // 16):
                sl = pl.ds(c8 * 16, 16)
                ri_v[r, sl] = ri_v[r, sl] - rj_v[r, sl]
            return rc

        lax.fori_loop(0, C, row, 0, unroll=2)
        pltpu.sync_copy(ri_v, out_hbm.at[pl.ds(base + g * C, C)])
        return carry

    lax.fori_loop(0, NCHUNK, chunk, 0)


@jax.jit
def _pix_sc(px, ii, jj):
    mesh = plsc.VectorSubcoreMesh(core_axis_name="c", subcore_axis_name="s")
    return pl.kernel(
        _sc_body,
        out_type=jax.ShapeDtypeStruct((B, D), jnp.float32),
        mesh=mesh,
        scratch_types=[
            pltpu.VMEM((NCHUNK, C), jnp.int32),
            pltpu.VMEM((NCHUNK, C), jnp.int32),
            pltpu.VMEM((C, D), jnp.float32),
            pltpu.VMEM((C, D), jnp.float32),
            pltpu.SemaphoreType.DMA,
            pltpu.SemaphoreType.DMA,
        ],
    )(px, ii, jj)


def kernel(px, idx_i, idx_j):
    ii = idx_i.astype(jnp.int32).reshape(NW, NCHUNK, C)
    jj = idx_j.astype(jnp.int32).reshape(NW, NCHUNK, C)
    return _pix_sc(px, ii, jj)


# speedup vs baseline: 5.5473x; 2.2576x over previous
import functools
import jax
import jax.numpy as jnp
from jax import lax
from jax.experimental import pallas as pl
from jax.experimental.pallas import tpu as pltpu
from jax.experimental.pallas import tpu_sc as plsc

B = 320000
D = 128
NC = 2
NS = 16
NW = NC * NS
EPW = B // NW
C = 80
NCHUNK = EPW // C
NBUF = 4


def _sc_body(px_hbm, ii_hbm, jj_hbm, out_hbm, ii_v, jj_v, ri, rj,
             gs0, gs1, gs2, gs3, ss0, ss1, ss2, ss3):
    gs = (gs0, gs1, gs2, gs3)
    ss = (ss0, ss1, ss2, ss3)
    wid = lax.axis_index("s") * NC + lax.axis_index("c")
    base = wid * EPW

    pltpu.sync_copy(ii_hbm.at[wid], ii_v)
    pltpu.sync_copy(jj_hbm.at[wid], jj_v)

    def start_gather(g, b):
        pltpu.async_copy(px_hbm.at[ii_v.at[g]], ri.at[b], gs[b])
        pltpu.async_copy(px_hbm.at[jj_v.at[g]], rj.at[b], gs[b])

    def wait_gather(b):
        pltpu.make_async_copy(px_hbm.at[ii_v.at[0]], ri.at[b], gs[b]).wait()
        pltpu.make_async_copy(px_hbm.at[jj_v.at[0]], rj.at[b], gs[b]).wait()

    def compute(b):
        def row(r, rc):
            for c8 in range(D // 16):
                sl = pl.ds(c8 * 16, 16)
                ri[b, r, sl] = ri[b, r, sl] - rj[b, r, sl]
            return rc
        lax.fori_loop(0, C, row, 0, unroll=2)

    def start_store(g, b):
        pltpu.async_copy(ri.at[b], out_hbm.at[pl.ds(base + g * C, C)], ss[b])

    def wait_store(b):
        pltpu.make_async_copy(ri.at[b], out_hbm.at[pl.ds(0, C)], ss[b]).wait()

    for b in range(NBUF - 1):
        start_gather(b, b)

    for b in range(NBUF):
        wait_gather(b)
        compute(b)
        start_store(b, b)
        if b == 0:
            start_gather(NBUF - 1, NBUF - 1)
        else:
            wait_store(b - 1)
            start_gather(b + NBUF - 1, b - 1)

    def group(p, carry):
        g0 = p * NBUF
        for b in range(NBUF):
            g = g0 + b
            bp = (b + NBUF - 1) % NBUF

            @pl.when(g < NCHUNK)
            def _():
                wait_gather(b)
                compute(b)
                start_store(g, b)

            @pl.when(g + NBUF - 1 < NCHUNK)
            def _():
                wait_store(bp)
                start_gather(g + NBUF - 1, bp)
        return carry

    lax.fori_loop(1, (NCHUNK + NBUF - 1) // NBUF, group, 0)

    for b in range(NBUF):
        wait_store(b)


@jax.jit
def _pix_sc(px, ii, jj):
    mesh = plsc.VectorSubcoreMesh(core_axis_name="c", subcore_axis_name="s")
    return pl.kernel(
        _sc_body,
        out_type=jax.ShapeDtypeStruct((B, D), jnp.float32),
        mesh=mesh,
        scratch_types=[
            pltpu.VMEM((NCHUNK, C), jnp.int32),
            pltpu.VMEM((NCHUNK, C), jnp.int32),
            pltpu.VMEM((NBUF, C, D), jnp.float32),
            pltpu.VMEM((NBUF, C, D), jnp.float32),
        ] + [pltpu.SemaphoreType.DMA] * (2 * NBUF),
    )(px, ii, jj)


def kernel(px, idx_i, idx_j):
    ii = idx_i.astype(jnp.int32).reshape(NW, NCHUNK, C)
    jj = idx_j.astype(jnp.int32).reshape(NW, NCHUNK, C)
    return _pix_sc(px, ii, jj)


# speedup vs baseline: 7.5477x; 1.3606x over previous
import functools
import jax
import jax.numpy as jnp
from jax import lax
from jax.experimental import pallas as pl
from jax.experimental.pallas import tpu as pltpu
from jax.experimental.pallas import tpu_sc as plsc

B = 320000
D = 128
NC = 2
NS = 16
NW = NC * NS
EPW = B // NW
C = 80
NCHUNK = EPW // C
NBUF = 4


def _sc_body(px_hbm, ii_hbm, jj_hbm, out_hbm, ii_v, jj_v, ri, rj,
             gs0, gs1, gs2, gs3, ss0, ss1, ss2, ss3):
    gs = (gs0, gs1, gs2, gs3)
    ss = (ss0, ss1, ss2, ss3)
    wid = lax.axis_index("s") * NC + lax.axis_index("c")
    base = wid * EPW

    pltpu.sync_copy(ii_hbm.at[wid], ii_v)
    pltpu.sync_copy(jj_hbm.at[wid], jj_v)

    def start_gather(g, b):
        pltpu.async_copy(px_hbm.at[ii_v.at[g]], ri.at[b], gs[b])
        pltpu.async_copy(px_hbm.at[jj_v.at[g]], rj.at[b], gs[b])

    def wait_gather(b):
        pltpu.make_async_copy(px_hbm.at[ii_v.at[0]], ri.at[b], gs[b]).wait()
        pltpu.make_async_copy(px_hbm.at[jj_v.at[0]], rj.at[b], gs[b]).wait()

    def compute(b):
        def row(r, rc):
            for c8 in range(D // 16):
                sl = pl.ds(c8 * 16, 16)
                plsc.addupdate(ri.at[b, r, sl], -rj[b, r, sl])
            return rc
        lax.fori_loop(0, C, row, 0, unroll=2)

    def start_store(g, b):
        pltpu.async_copy(ri.at[b], out_hbm.at[pl.ds(base + g * C, C)], ss[b])

    def wait_store(b):
        pltpu.make_async_copy(ri.at[b], out_hbm.at[pl.ds(0, C)], ss[b]).wait()

    for b in range(NBUF - 1):
        start_gather(b, b)

    for b in range(NBUF):
        wait_gather(b)
        compute(b)
        start_store(b, b)
        if b == 0:
            start_gather(NBUF - 1, NBUF - 1)
        else:
            wait_store(b - 1)
            start_gather(b + NBUF - 1, b - 1)

    def group(p, carry):
        g0 = p * NBUF
        for b in range(NBUF):
            g = g0 + b
            bp = (b + NBUF - 1) % NBUF

            @pl.when(g < NCHUNK)
            def _():
                wait_gather(b)
                compute(b)
                start_store(g, b)

            @pl.when(g + NBUF - 1 < NCHUNK)
            def _():
                wait_store(bp)
                start_gather(g + NBUF - 1, bp)
        return carry

    lax.fori_loop(1, (NCHUNK + NBUF - 1) // NBUF, group, 0)

    for b in range(NBUF):
        wait_store(b)


@jax.jit
def _pix_sc(px, ii, jj):
    mesh = plsc.VectorSubcoreMesh(core_axis_name="c", subcore_axis_name="s")
    return pl.kernel(
        _sc_body,
        out_type=jax.ShapeDtypeStruct((B, D), jnp.float32),
        mesh=mesh,
        scratch_types=[
            pltpu.VMEM((NCHUNK, C), jnp.int32),
            pltpu.VMEM((NCHUNK, C), jnp.int32),
            pltpu.VMEM((NBUF, C, D), jnp.float32),
            pltpu.VMEM((NBUF, C, D), jnp.float32),
        ] + [pltpu.SemaphoreType.DMA] * (2 * NBUF),
    )(px, ii, jj)


def kernel(px, idx_i, idx_j):
    ii = idx_i.astype(jnp.int32).reshape(NW, NCHUNK, C)
    jj = idx_j.astype(jnp.int32).reshape(NW, NCHUNK, C)
    return _pix_sc(px, ii, jj)


# speedup vs baseline: 7.6615x; 1.0151x over previous
import functools
import jax
import jax.numpy as jnp
from jax import lax
from jax.experimental import pallas as pl
from jax.experimental.pallas import tpu as pltpu
from jax.experimental.pallas import tpu_sc as plsc

B = 320000
D = 128
NC = 2
NS = 16
NW = NC * NS
EPW = B // NW
C = 80
NCHUNK = EPW // C
NBUF = 4


def _sc_body(px_hbm, ii_hbm, jj_hbm, out_hbm, ii_v, jj_v, ri, rj,
             gs0, gs1, gs2, gs3, ss0, ss1, ss2, ss3):
    gs = (gs0, gs1, gs2, gs3)
    ss = (ss0, ss1, ss2, ss3)
    sid = lax.axis_index("s")
    wid = sid * NC + lax.axis_index("c")
    base = wid * EPW

    pltpu.sync_copy(ii_hbm.at[pl.ds(base, EPW)], ii_v)
    pltpu.sync_copy(jj_hbm.at[pl.ds(base, EPW)], jj_v)

    def start_gather(g, b):
        pltpu.async_copy(px_hbm.at[ii_v.at[pl.ds(g * C, C)]], ri.at[b], gs[b])
        pltpu.async_copy(px_hbm.at[jj_v.at[pl.ds(g * C, C)]], rj.at[b], gs[b])

    def wait_gather(b):
        pltpu.make_async_copy(px_hbm.at[ii_v.at[pl.ds(0, C)]], ri.at[b], gs[b]).wait()
        pltpu.make_async_copy(px_hbm.at[jj_v.at[pl.ds(0, C)]], rj.at[b], gs[b]).wait()

    def compute(b):
        def row(r, rc):
            for c8 in range(D // 16):
                sl = pl.ds(c8 * 16, 16)
                plsc.addupdate(ri.at[b, r, sl], -rj[b, r, sl])
            return rc
        lax.fori_loop(0, C, row, 0, unroll=4)

    def start_store(g, b):
        pltpu.async_copy(ri.at[b], out_hbm.at[pl.ds(base + g * C, C)], ss[b])

    def wait_store(b):
        pltpu.make_async_copy(ri.at[b], out_hbm.at[pl.ds(0, C)], ss[b]).wait()

    for b in range(NBUF - 1):
        start_gather(b, b)

    for b in range(NBUF):
        wait_gather(b)
        compute(b)
        start_store(b, b)
        if b == 0:
            start_gather(NBUF - 1, NBUF - 1)
        else:
            wait_store(b - 1)
            start_gather(b + NBUF - 1, b - 1)

    def group(p, carry):
        g0 = p * NBUF
        for b in range(NBUF):
            g = g0 + b
            bp = (b + NBUF - 1) % NBUF

            @pl.when(g < NCHUNK)
            def _():
                wait_gather(b)
                compute(b)
                start_store(g, b)

            @pl.when(g + NBUF - 1 < NCHUNK)
            def _():
                wait_store(bp)
                start_gather(g + NBUF - 1, bp)
        return carry

    lax.fori_loop(1, (NCHUNK + NBUF - 1) // NBUF, group, 0)

    for b in range(NBUF):
        wait_store(b)


@jax.jit
def _pix_sc(px, ii, jj):
    mesh = plsc.VectorSubcoreMesh(core_axis_name="c", subcore_axis_name="s")
    return pl.kernel(
        _sc_body,
        out_type=jax.ShapeDtypeStruct((B, D), jnp.float32),
        mesh=mesh,
        scratch_types=[
            pltpu.VMEM((EPW,), jnp.int32),
            pltpu.VMEM((EPW,), jnp.int32),
            pltpu.VMEM((NBUF, C, D), jnp.float32),
            pltpu.VMEM((NBUF, C, D), jnp.float32),
        ] + [pltpu.SemaphoreType.DMA] * (2 * NBUF),
    )(px, ii, jj)


def kernel(px, idx_i, idx_j):
    return _pix_sc(px, idx_i.astype(jnp.int32), idx_j.astype(jnp.int32))


# speedup vs baseline: 7.6658x; 1.0006x over previous
import functools
import jax
import jax.numpy as jnp
from jax import lax
from jax.experimental import pallas as pl
from jax.experimental.pallas import tpu as pltpu
from jax.experimental.pallas import tpu_sc as plsc

B = 320000
D = 128
NC = 2
NS = 16
NW = NC * NS
EPW = B // NW
C = 80
NCHUNK = EPW // C
NBUF = 5
LOOK = 3


def _sc_body(px_hbm, ii_hbm, jj_hbm, out_hbm, ii_v, jj_v, ri, rj,
             gs0, gs1, gs2, gs3, gs4, ss0, ss1, ss2, ss3, ss4):
    gs = (gs0, gs1, gs2, gs3, gs4)
    ss = (ss0, ss1, ss2, ss3, ss4)
    wid = lax.axis_index("s") * NC + lax.axis_index("c")
    base = wid * EPW

    pltpu.sync_copy(ii_hbm.at[pl.ds(base, EPW)], ii_v)
    pltpu.sync_copy(jj_hbm.at[pl.ds(base, EPW)], jj_v)

    def start_gather(g, b):
        pltpu.async_copy(px_hbm.at[ii_v.at[pl.ds(g * C, C)]], ri.at[b], gs[b])
        pltpu.async_copy(px_hbm.at[jj_v.at[pl.ds(g * C, C)]], rj.at[b], gs[b])

    def wait_gather(b):
        pltpu.make_async_copy(px_hbm.at[ii_v.at[pl.ds(0, C)]], ri.at[b], gs[b]).wait()
        pltpu.make_async_copy(px_hbm.at[jj_v.at[pl.ds(0, C)]], rj.at[b], gs[b]).wait()

    def compute(b):
        def row(r, rc):
            for c8 in range(D // 16):
                sl = pl.ds(c8 * 16, 16)
                plsc.addupdate(ri.at[b, r, sl], -rj[b, r, sl])
            return rc
        lax.fori_loop(0, C, row, 0, unroll=4)

    def start_store(g, b):
        pltpu.async_copy(ri.at[b], out_hbm.at[pl.ds(base + g * C, C)], ss[b])

    def wait_store(b):
        pltpu.make_async_copy(ri.at[b], out_hbm.at[pl.ds(0, C)], ss[b]).wait()

    for k in range(LOOK):
        start_gather(k, k)

    for g in range(NBUF):
        b = g % NBUF
        wait_gather(b)
        compute(b)
        start_store(g, b)
        t = g + LOOK
        bq = t % NBUF
        if t >= NBUF:
            wait_store(bq)
        start_gather(t, bq)

    def group(p, carry):
        g0 = p * NBUF
        for b in range(NBUF):
            g = g0 + b
            bq = (b + LOOK) % NBUF

            @pl.when(g < NCHUNK)
            def _():
                wait_gather(b)
                compute(b)
                start_store(g, b)

            @pl.when(g + LOOK < NCHUNK)
            def _():
                wait_store(bq)
                start_gather(g + LOOK, bq)
        return carry

    lax.fori_loop(1, (NCHUNK + NBUF - 1) // NBUF, group, 0)

    for b in range(NBUF):
        wait_store(b)


@jax.jit
def _pix_sc(px, ii, jj):
    mesh = plsc.VectorSubcoreMesh(core_axis_name="c", subcore_axis_name="s")
    return pl.kernel(
        _sc_body,
        out_type=jax.ShapeDtypeStruct((B, D), jnp.float32),
        mesh=mesh,
        scratch_types=[
            pltpu.VMEM((EPW,), jnp.int32),
            pltpu.VMEM((EPW,), jnp.int32),
            pltpu.VMEM((NBUF, C, D), jnp.float32),
            pltpu.VMEM((NBUF, C, D), jnp.float32),
        ] + [pltpu.SemaphoreType.DMA] * (2 * NBUF),
    )(px, ii, jj)


def kernel(px, idx_i, idx_j):
    return _pix_sc(px, idx_i.astype(jnp.int32), idx_j.astype(jnp.int32))
